# Initial kernel scaffold; baseline (speedup 1.0000x reference)
import jax
import jax.numpy as jnp
from jax.experimental import pallas as pl


def kernel(n_features, e_features, edge_index, W_apply_w, W_apply_b, W_edge_w, W_edge_b, disc_W):
    raise NotImplementedError("write your pallas kernel here")


# speedup vs baseline: 12.3004x; 12.3004x over previous
import numpy as np
import jax
import jax.numpy as jnp
from jax import lax
from jax.experimental import pallas as pl
from jax.experimental.pallas import tpu as pltpu
from jax.experimental.pallas import tpu_sc as plsc

_N = 10000
_E = 320000
_DIN = 128
_EDIM = 16
_H = 128
_EOUT = 256

_NC = 2
_NS = 16
_NW = _NC * _NS
_NP = 10240
_ROWS_PT = _NP // _NS
_EPAD = 327680
_EPW_PAD = _EPAD // _NW
_BLK = 1024
_NBLK = _EPW_PAD // _BLK
_EPW = _E // _NW
_TN = 640

def _make_perm_pad():
    try:
        try:
            dev = jax.local_devices(backend="cpu")[0]
            with jax.default_device(dev):
                p = np.asarray(jax.random.permutation(jax.random.key(1), _E))
        except Exception:
            p = np.asarray(jax.random.permutation(jax.random.key(1), _E))
    except Exception:
        p = np.arange(_E)
    return np.concatenate([p.astype(np.int32),
                           np.arange(_E, _EPAD, dtype=np.int32)])


_PERM_PAD = _make_perm_pad()


def _perm_pad():
    return _PERM_PAD

def _sc_mesh():
    return plsc.VectorSubcoreMesh(core_axis_name="c", subcore_axis_name="s",
                                  num_cores=_NC, num_subcores=_NS)


def _sc_scatter_body(efeat, src, dst, perm, zeros_h, ones_h,
                     spos_o, sneg_o, din_o, dout_o,
                     feat_v, featp_v, ones_v, zeros_v, src_v, dst_v, perm_v,
                     acc_pos, acc_neg, acc_din, acc_dout, sem):
    cid = lax.axis_index("c")
    sid = lax.axis_index("s")
    wid = cid * _NS + sid

    pltpu.sync_copy(zeros_h, zeros_v)
    pltpu.sync_copy(ones_h, ones_v)

    row0 = sid * _ROWS_PT
    pltpu.sync_copy(zeros_v, acc_pos.at[pl.ds(row0, _ROWS_PT)])
    pltpu.sync_copy(zeros_v, acc_neg.at[pl.ds(row0, _ROWS_PT)])
    pltpu.sync_copy(zeros_v, acc_din.at[pl.ds(row0, _ROWS_PT)])
    pltpu.sync_copy(zeros_v, acc_dout.at[pl.ds(row0, _ROWS_PT)])
    plsc.subcore_barrier()

    def block(b, carry):
        base = wid * _EPW_PAD + b * _BLK
        pltpu.sync_copy(efeat.at[pl.ds(base, _BLK)], feat_v)
        pltpu.sync_copy(src.at[pl.ds(base, _BLK)], src_v)
        pltpu.sync_copy(dst.at[pl.ds(base, _BLK)], dst_v)
        pltpu.sync_copy(perm.at[pl.ds(base, _BLK)], perm_v)
        pltpu.async_copy(efeat.at[perm_v], featp_v, sem).wait()
        pltpu.sync_copy(feat_v, acc_pos.at[dst_v], add=True)
        pltpu.sync_copy(featp_v, acc_neg.at[dst_v], add=True)
        pltpu.sync_copy(ones_v, acc_din.at[dst_v], add=True)
        pltpu.sync_copy(ones_v, acc_dout.at[src_v], add=True)
        return carry

    lax.fori_loop(0, _NBLK, block, 0)
    plsc.subcore_barrier()

    rows = pl.ds(row0, _ROWS_PT)
    pltpu.sync_copy(acc_pos.at[rows], spos_o.at[cid, rows])
    pltpu.sync_copy(acc_neg.at[rows], sneg_o.at[cid, rows])
    pltpu.sync_copy(acc_din.at[rows], din_o.at[cid, rows])
    pltpu.sync_copy(acc_dout.at[rows], dout_o.at[cid, rows])


def _run_sc_scatter(ef, src_p, dst_p, perm_p):
    acc = jax.ShapeDtypeStruct((_NC, _NP, _EDIM), jnp.float32)
    f = pl.kernel(
        _sc_scatter_body,
        out_type=(acc, acc, acc, acc),
        mesh=_sc_mesh(),
        compiler_params=pltpu.CompilerParams(use_tc_tiling_on_sc=False),
        scratch_types=[
            pltpu.VMEM((_BLK, _EDIM), jnp.float32),
            pltpu.VMEM((_BLK, _EDIM), jnp.float32),
            pltpu.VMEM((_BLK, _EDIM), jnp.float32),
            pltpu.VMEM((_ROWS_PT, _EDIM), jnp.float32),
            pltpu.VMEM((_BLK,), jnp.int32),
            pltpu.VMEM((_BLK,), jnp.int32),
            pltpu.VMEM((_BLK,), jnp.int32),
            pltpu.VMEM_SHARED((_NP, _EDIM), jnp.float32),
            pltpu.VMEM_SHARED((_NP, _EDIM), jnp.float32),
            pltpu.VMEM_SHARED((_NP, _EDIM), jnp.float32),
            pltpu.VMEM_SHARED((_NP, _EDIM), jnp.float32),
            pltpu.SemaphoreType.DMA,
        ],
    )
    zeros_h = jnp.zeros((_ROWS_PT, _EDIM), jnp.float32)
    ones_h = jnp.ones((_BLK, _EDIM), jnp.float32)
    return f(ef, src_p, dst_p, perm_p, zeros_h, ones_h)


def _b1_body(nf_ref, spos_ref, sneg_ref, din_ref, wnT_ref, weT_ref, b_ref,
             hp_ref, hn_ref):
    din = din_ref[0, :, 0:1] + din_ref[1, :, 0:1]
    inv = 1.0 / jnp.maximum(din, 1.0)
    sp = (spos_ref[0] + spos_ref[1]) * inv
    sn = (sneg_ref[0] + sneg_ref[1]) * inv
    base = jnp.dot(nf_ref[...], wnT_ref[...],
                   preferred_element_type=jnp.float32) + b_ref[...]
    weT = weT_ref[...]
    hp_ref[...] = jnp.maximum(
        base + jnp.dot(sp, weT, preferred_element_type=jnp.float32), 0.0)
    hn_ref[...] = jnp.maximum(
        base + jnp.dot(sn, weT, preferred_element_type=jnp.float32), 0.0)


def _run_b1(nf, spos, sneg, din, wnT, weT, brow):
    grid = (_NP // _TN,)
    seg = pl.BlockSpec((2, _TN, _EDIM), lambda i: (0, i, 0))
    out = jax.ShapeDtypeStruct((_NP, _H), jnp.float32)
    return pl.pallas_call(
        _b1_body,
        grid=grid,
        in_specs=[
            pl.BlockSpec((_TN, _DIN), lambda i: (i, 0)),
            seg, seg, seg,
            pl.BlockSpec((_DIN, _H), lambda i: (0, 0)),
            pl.BlockSpec((_EDIM, _H), lambda i: (0, 0)),
            pl.BlockSpec((1, _H), lambda i: (0, 0)),
        ],
        out_specs=[pl.BlockSpec((_TN, _H), lambda i: (i, 0)),
                   pl.BlockSpec((_TN, _H), lambda i: (i, 0))],
        out_shape=[out, out],
    )(nf, spos, sneg, din, wnT, weT, brow)


def _b2_body(hp_ref, din_ref, dout_ref, wew_ref, beb_ref, dw_ref,
             u_ref, c_ref):
    mask = (lax.broadcasted_iota(jnp.int32, (_NP, 1), 0) < _N).astype(
        jnp.float32)
    din = (din_ref[0, :, 0:1] + din_ref[1, :, 0:1]) * mask
    dout = (dout_ref[0, :, 0:1] + dout_ref[1, :, 0:1]) * mask
    hp = hp_ref[...]
    msrc = jnp.sum(hp * dout, axis=0, keepdims=True)
    mdst = jnp.sum(hp * din, axis=0, keepdims=True)
    wew = wew_ref[...]
    w1 = wew[:, :_H]
    w2 = wew[:, _H:]
    dims = (((1,), (1,)), ((), ()))
    me = (lax.dot_general(msrc, w1, dims, preferred_element_type=jnp.float32)
          + lax.dot_general(mdst, w2, dims, preferred_element_type=jnp.float32)
          ) * (1.0 / _E) + beb_ref[...]
    summ = jax.nn.sigmoid(me)
    ws = lax.dot_general(summ, dw_ref[...], dims,
                         preferred_element_type=jnp.float32)
    dims2 = (((1,), (0,)), ((), ()))
    u1 = lax.dot_general(ws, w1, dims2, preferred_element_type=jnp.float32)
    u2 = lax.dot_general(ws, w2, dims2, preferred_element_type=jnp.float32)
    u_ref[...] = jnp.concatenate([u1, u2], axis=0)
    c_ref[...] = jnp.sum(beb_ref[...] * ws).reshape(1, 1)


def _run_b2(h_pos, din, dout, wew, beb_row, dw):
    return pl.pallas_call(
        _b2_body,
        out_shape=[jax.ShapeDtypeStruct((2, _H), jnp.float32),
                   jax.ShapeDtypeStruct((1, 1), jnp.float32)],
    )(h_pos, din, dout, wew, beb_row, dw)


def _b3_body(hp_ref, hn_ref, u_ref, op_ref, on_ref):
    u = u_ref[...]
    dims = (((1,), (1,)), ((), ()))
    op_ref[...] = lax.dot_general(u, hp_ref[...], dims,
                                  preferred_element_type=jnp.float32)
    on_ref[...] = lax.dot_general(u, hn_ref[...], dims,
                                  preferred_element_type=jnp.float32)


def _run_b3(h_pos, h_neg, u):
    grid = (_NP // _TN,)
    out = jax.ShapeDtypeStruct((2, _NP), jnp.float32)
    return pl.pallas_call(
        _b3_body,
        grid=grid,
        in_specs=[
            pl.BlockSpec((_TN, _H), lambda i: (i, 0)),
            pl.BlockSpec((_TN, _H), lambda i: (i, 0)),
            pl.BlockSpec((2, _H), lambda i: (0, 0)),
        ],
        out_specs=[pl.BlockSpec((2, _TN), lambda i: (0, i)),
                   pl.BlockSpec((2, _TN), lambda i: (0, i))],
        out_shape=[out, out],
    )(h_pos, h_neg, u)


def _sc_edge_body(src, dst, pp, qp, pn, qn, xp_o, xn_o,
                  pp_v, qp_v, pn_v, qn_v, src_v, dst_v, xp_v, xn_v):
    cid = lax.axis_index("c")
    sid = lax.axis_index("s")
    wid = cid * _NS + sid

    pltpu.sync_copy(pp, pp_v)
    pltpu.sync_copy(qp, qp_v)
    pltpu.sync_copy(pn, pn_v)
    pltpu.sync_copy(qn, qn_v)
    base = wid * _EPW
    pltpu.sync_copy(src.at[pl.ds(base, _EPW)], src_v)
    pltpu.sync_copy(dst.at[pl.ds(base, _EPW)], dst_v)

    def it(i, carry):
        s = pl.ds(i * 16, 16)
        sv = src_v[s]
        dv = dst_v[s]
        xp_v[s] = (plsc.load_gather(pp_v, [sv])
                   + plsc.load_gather(qp_v, [dv]))
        xn_v[s] = (plsc.load_gather(pn_v, [sv])
                   + plsc.load_gather(qn_v, [dv]))
        return carry

    lax.fori_loop(0, _EPW // 16, it, 0)
    pltpu.sync_copy(xp_v, xp_o.at[pl.ds(base, _EPW)])
    pltpu.sync_copy(xn_v, xn_o.at[pl.ds(base, _EPW)])


def _run_sc_edge(src, dst, pp, qp, pn, qn):
    out = jax.ShapeDtypeStruct((_E,), jnp.float32)
    f = pl.kernel(
        _sc_edge_body,
        out_type=(out, out),
        mesh=_sc_mesh(),
        compiler_params=pltpu.CompilerParams(use_tc_tiling_on_sc=False,
                                             needs_layout_passes=False),
        scratch_types=[
            pltpu.VMEM((_NP,), jnp.float32),
            pltpu.VMEM((_NP,), jnp.float32),
            pltpu.VMEM((_NP,), jnp.float32),
            pltpu.VMEM((_NP,), jnp.float32),
            pltpu.VMEM((_EPW,), jnp.int32),
            pltpu.VMEM((_EPW,), jnp.int32),
            pltpu.VMEM((_EPW,), jnp.float32),
            pltpu.VMEM((_EPW,), jnp.float32),
        ],
    )
    return f(src, dst, pp, qp, pn, qn)


def _d_body(xp_ref, xn_ref, c_ref, o_ref):
    c = c_ref[0, 0]
    lp = jnp.mean(jax.nn.softplus(-(xp_ref[...] + c)))
    ln = jnp.mean(jax.nn.softplus(xn_ref[...] + c))
    o_ref[...] = (lp + ln).reshape(1, 1)


def _run_d(xp2, xn2, c):
    return pl.pallas_call(
        _d_body,
        out_shape=jax.ShapeDtypeStruct((1, 1), jnp.float32),
    )(xp2, xn2, c)


def kernel(n_features, e_features, edge_index, W_apply_w, W_apply_b,
           W_edge_w, W_edge_b, disc_W):
    nf = jnp.concatenate(
        [n_features.reshape(_N, _DIN),
         jnp.zeros((_NP - _N, _DIN), jnp.float32)], axis=0)
    ef = jnp.concatenate(
        [e_features.reshape(_E, _EDIM),
         jnp.zeros((_EPAD - _E, _EDIM), jnp.float32)], axis=0)
    src = edge_index[0]
    dst = edge_index[1]
    padi = jnp.full((_EPAD - _E,), _N, jnp.int32)
    src_p = jnp.concatenate([src, padi])
    dst_p = jnp.concatenate([dst, padi])
    perm_p = jnp.asarray(_perm_pad())

    spos, sneg, din, dout = _run_sc_scatter(ef, src_p, dst_p, perm_p)

    wnT = W_apply_w[:, :_DIN].T
    weT = W_apply_w[:, _DIN:].T
    brow = W_apply_b.reshape(1, _H)
    h_pos, h_neg = _run_b1(nf, spos, sneg, din, wnT, weT, brow)

    beb_row = W_edge_b.reshape(1, _EOUT)
    u, c = _run_b2(h_pos, din, dout, W_edge_w, beb_row, disc_W)

    opos, oneg = _run_b3(h_pos, h_neg, u)

    xp, xn = _run_sc_edge(src, dst, opos[0], opos[1], oneg[0], oneg[1])

    loss = _run_d(xp.reshape(_E // _DIN, _DIN), xn.reshape(_E // _DIN, _DIN),
                  c)
    return loss[0, 0]


# speedup vs baseline: 17.6579x; 1.4356x over previous
import numpy as np
import jax
import jax.numpy as jnp
from jax import lax
from jax.experimental import pallas as pl
from jax.experimental.pallas import tpu as pltpu
from jax.experimental.pallas import tpu_sc as plsc

_N = 10000
_E = 320000
_DIN = 128
_EDIM = 16
_H = 128
_EOUT = 256

_NC = 2
_NS = 16
_NW = _NC * _NS
_NP = 10240
_ROWS_PT = _NP // _NS
_EPW = _E // _NW
_BLK = 1000
_NBLK = _EPW // _BLK
_TN = 640


def _make_perm():
    try:
        try:
            dev = jax.local_devices(backend="cpu")[0]
            with jax.default_device(dev):
                p = np.asarray(jax.random.permutation(jax.random.key(1), _E))
        except Exception:
            p = np.asarray(jax.random.permutation(jax.random.key(1), _E))
    except Exception:
        p = np.arange(_E)
    return p.astype(np.int32)


_PERM = _make_perm()


def _sc_mesh():
    return plsc.VectorSubcoreMesh(core_axis_name="c", subcore_axis_name="s",
                                  num_cores=_NC, num_subcores=_NS)


def _sc_scatter_body(efeat, src, dst, perm, zeros_h, ones_h,
                     spos_o, sneg_o, din_o, dout_o,
                     feat_v, featp_v, ones_v, zeros_v, src_v, dst_v, perm_v,
                     acc_pos, acc_neg, acc_din, acc_dout, sem):
    cid = lax.axis_index("c")
    sid = lax.axis_index("s")
    wid = cid * _NS + sid

    pltpu.sync_copy(zeros_h, zeros_v)
    pltpu.sync_copy(ones_h, ones_v)

    row0 = sid * _ROWS_PT
    pltpu.sync_copy(zeros_v, acc_pos.at[pl.ds(row0, _ROWS_PT)])
    pltpu.sync_copy(zeros_v, acc_neg.at[pl.ds(row0, _ROWS_PT)])
    pltpu.sync_copy(zeros_v, acc_din.at[pl.ds(row0, _ROWS_PT)])
    pltpu.sync_copy(zeros_v, acc_dout.at[pl.ds(row0, _ROWS_PT)])
    plsc.subcore_barrier()

    def block(b, carry):
        base = wid * _EPW + b * _BLK
        pltpu.sync_copy(efeat.at[pl.ds(base, _BLK)], feat_v)
        pltpu.sync_copy(src.at[pl.ds(base, _BLK)], src_v)
        pltpu.sync_copy(dst.at[pl.ds(base, _BLK)], dst_v)
        pltpu.sync_copy(perm.at[pl.ds(base, _BLK)], perm_v)
        pltpu.async_copy(efeat.at[perm_v], featp_v, sem).wait()
        pltpu.sync_copy(feat_v, acc_pos.at[dst_v], add=True)
        pltpu.sync_copy(featp_v, acc_neg.at[dst_v], add=True)
        pltpu.sync_copy(ones_v, acc_din.at[dst_v], add=True)
        pltpu.sync_copy(ones_v, acc_dout.at[src_v], add=True)
        return carry

    lax.fori_loop(0, _NBLK, block, 0)
    plsc.subcore_barrier()

    rows = pl.ds(row0, _ROWS_PT)
    pltpu.sync_copy(acc_pos.at[rows], spos_o.at[cid, rows])
    pltpu.sync_copy(acc_neg.at[rows], sneg_o.at[cid, rows])
    pltpu.sync_copy(acc_din.at[rows], din_o.at[cid, rows])
    pltpu.sync_copy(acc_dout.at[rows], dout_o.at[cid, rows])


def _run_sc_scatter(ef, src, dst, perm):
    acc = jax.ShapeDtypeStruct((_NC, _NP, _EDIM), jnp.float32)
    f = pl.kernel(
        _sc_scatter_body,
        out_type=(acc, acc, acc, acc),
        mesh=_sc_mesh(),
        compiler_params=pltpu.CompilerParams(use_tc_tiling_on_sc=False),
        scratch_types=[
            pltpu.VMEM((_BLK, _EDIM), jnp.float32),
            pltpu.VMEM((_BLK, _EDIM), jnp.float32),
            pltpu.VMEM((_BLK, _EDIM), jnp.float32),
            pltpu.VMEM((_ROWS_PT, _EDIM), jnp.float32),
            pltpu.VMEM((_BLK,), jnp.int32),
            pltpu.VMEM((_BLK,), jnp.int32),
            pltpu.VMEM((_BLK,), jnp.int32),
            pltpu.VMEM_SHARED((_NP, _EDIM), jnp.float32),
            pltpu.VMEM_SHARED((_NP, _EDIM), jnp.float32),
            pltpu.VMEM_SHARED((_NP, _EDIM), jnp.float32),
            pltpu.VMEM_SHARED((_NP, _EDIM), jnp.float32),
            pltpu.SemaphoreType.DMA,
        ],
    )
    zeros_h = jnp.zeros((_ROWS_PT, _EDIM), jnp.float32)
    ones_h = jnp.ones((_BLK, _EDIM), jnp.float32)
    return f(ef, src, dst, perm, zeros_h, ones_h)


def _b1_body(nf_ref, spos_ref, sneg_ref, din_ref, dout_ref,
             wnT_ref, weT_ref, b_ref, wew_ref, beb_ref, dw_ref,
             hp_ref, hn_ref, u_ref, c_ref, acc_ref):
    i = pl.program_id(0)
    din = din_ref[0, :, 0:1] + din_ref[1, :, 0:1]
    dout = dout_ref[0, :, 0:1] + dout_ref[1, :, 0:1]
    inv = 1.0 / jnp.maximum(din, 1.0)
    sp = (spos_ref[0] + spos_ref[1]) * inv
    sn = (sneg_ref[0] + sneg_ref[1]) * inv
    base = jnp.dot(nf_ref[...], wnT_ref[...],
                   preferred_element_type=jnp.float32) + b_ref[...]
    weT = weT_ref[...]
    hp = jnp.maximum(
        base + jnp.dot(sp, weT, preferred_element_type=jnp.float32), 0.0)
    hn = jnp.maximum(
        base + jnp.dot(sn, weT, preferred_element_type=jnp.float32), 0.0)
    hp_ref[...] = hp
    hn_ref[...] = hn

    mask = ((lax.broadcasted_iota(jnp.int32, (_TN, 1), 0) + i * _TN)
            < _N).astype(jnp.float32)
    msrc_t = jnp.sum(hp * (dout * mask), axis=0, keepdims=True)
    mdst_t = jnp.sum(hp * (din * mask), axis=0, keepdims=True)

    @pl.when(i == 0)
    def _init():
        acc_ref[...] = jnp.zeros((2, _H), jnp.float32)

    acc_ref[0:1, :] += msrc_t
    acc_ref[1:2, :] += mdst_t

    @pl.when(i == (_NP // _TN) - 1)
    def _fin():
        msrc = acc_ref[0:1, :]
        mdst = acc_ref[1:2, :]
        wew = wew_ref[...]
        w1 = wew[:, :_H]
        w2 = wew[:, _H:]
        dims = (((1,), (1,)), ((), ()))
        me = (lax.dot_general(msrc, w1, dims,
                              preferred_element_type=jnp.float32)
              + lax.dot_general(mdst, w2, dims,
                                preferred_element_type=jnp.float32)
              ) * (1.0 / _E) + beb_ref[...]
        summ = jax.nn.sigmoid(me)
        ws = lax.dot_general(summ, dw_ref[...], dims,
                             preferred_element_type=jnp.float32)
        dims2 = (((1,), (0,)), ((), ()))
        u1 = lax.dot_general(ws, w1, dims2, preferred_element_type=jnp.float32)
        u2 = lax.dot_general(ws, w2, dims2, preferred_element_type=jnp.float32)
        u_ref[...] = jnp.concatenate([u1, u2], axis=0)
        c_ref[...] = jnp.sum(beb_ref[...] * ws).reshape(1, 1)


def _run_b1(nf, spos, sneg, din, dout, wnT, weT, brow, wew, beb_row, dw):
    grid = (_NP // _TN,)
    seg = pl.BlockSpec((2, _TN, _EDIM), lambda i: (0, i, 0))
    hout = jax.ShapeDtypeStruct((_NP, _H), jnp.float32)
    return pl.pallas_call(
        _b1_body,
        grid=grid,
        in_specs=[
            pl.BlockSpec((_TN, _DIN), lambda i: (i, 0)),
            seg, seg, seg, seg,
            pl.BlockSpec((_DIN, _H), lambda i: (0, 0)),
            pl.BlockSpec((_EDIM, _H), lambda i: (0, 0)),
            pl.BlockSpec((1, _H), lambda i: (0, 0)),
            pl.BlockSpec((_EOUT, _EOUT), lambda i: (0, 0)),
            pl.BlockSpec((1, _EOUT), lambda i: (0, 0)),
            pl.BlockSpec((_EOUT, _EOUT), lambda i: (0, 0)),
        ],
        out_specs=[pl.BlockSpec((_TN, _H), lambda i: (i, 0)),
                   pl.BlockSpec((_TN, _H), lambda i: (i, 0)),
                   pl.BlockSpec((2, _H), lambda i: (0, 0)),
                   pl.BlockSpec((1, 1), lambda i: (0, 0))],
        out_shape=[hout, hout,
                   jax.ShapeDtypeStruct((2, _H), jnp.float32),
                   jax.ShapeDtypeStruct((1, 1), jnp.float32)],
        scratch_shapes=[pltpu.VMEM((2, _H), jnp.float32)],
    )(nf, spos, sneg, din, dout, wnT, weT, brow, wew, beb_row, dw)


def _b3_body(hp_ref, hn_ref, u_ref, o_ref):
    u = u_ref[...]
    dims = (((1,), (1,)), ((), ()))
    pq_p = lax.dot_general(u, hp_ref[...], dims,
                           preferred_element_type=jnp.float32)
    pq_n = lax.dot_general(u, hn_ref[...], dims,
                           preferred_element_type=jnp.float32)
    o_ref[...] = jnp.concatenate([pq_p, pq_n], axis=0)


def _run_b3(h_pos, h_neg, u):
    grid = (_NP // _TN,)
    return pl.pallas_call(
        _b3_body,
        grid=grid,
        in_specs=[
            pl.BlockSpec((_TN, _H), lambda i: (i, 0)),
            pl.BlockSpec((_TN, _H), lambda i: (i, 0)),
            pl.BlockSpec((2, _H), lambda i: (0, 0)),
        ],
        out_specs=pl.BlockSpec((4, _TN), lambda i: (0, i)),
        out_shape=jax.ShapeDtypeStruct((4, _NP), jnp.float32),
    )(h_pos, h_neg, u)


def _sc_edge_body(src, dst, tab, xp_o, xn_o,
                  tab_v, src_v, dst_v, xp_v, xn_v):
    cid = lax.axis_index("c")
    sid = lax.axis_index("s")
    wid = cid * _NS + sid

    pltpu.sync_copy(tab, tab_v)
    base = wid * _EPW
    pltpu.sync_copy(src.at[pl.ds(base, _EPW)], src_v)
    pltpu.sync_copy(dst.at[pl.ds(base, _EPW)], dst_v)

    def it(i, carry):
        s = pl.ds(i * 16, 16)
        sv = src_v[s]
        dv = dst_v[s]
        xp_v[s] = (plsc.load_gather(tab_v, [sv])
                   + plsc.load_gather(tab_v, [dv + _NP]))
        xn_v[s] = (plsc.load_gather(tab_v, [sv + 2 * _NP])
                   + plsc.load_gather(tab_v, [dv + 3 * _NP]))
        return carry

    lax.fori_loop(0, _EPW // 16, it, 0)
    pltpu.sync_copy(xp_v, xp_o.at[pl.ds(base, _EPW)])
    pltpu.sync_copy(xn_v, xn_o.at[pl.ds(base, _EPW)])


def _run_sc_edge(src, dst, tab):
    out = jax.ShapeDtypeStruct((_E,), jnp.float32)
    f = pl.kernel(
        _sc_edge_body,
        out_type=(out, out),
        mesh=_sc_mesh(),
        compiler_params=pltpu.CompilerParams(use_tc_tiling_on_sc=False,
                                             needs_layout_passes=False),
        scratch_types=[
            pltpu.VMEM((4 * _NP,), jnp.float32),
            pltpu.VMEM((_EPW,), jnp.int32),
            pltpu.VMEM((_EPW,), jnp.int32),
            pltpu.VMEM((_EPW,), jnp.float32),
            pltpu.VMEM((_EPW,), jnp.float32),
        ],
    )
    return f(src, dst, tab)


def _d_body(xp_ref, xn_ref, c_ref, o_ref):
    c = c_ref[0, 0]
    lp = jnp.mean(jax.nn.softplus(-(xp_ref[...] + c)))
    ln = jnp.mean(jax.nn.softplus(xn_ref[...] + c))
    o_ref[...] = (lp + ln).reshape(1, 1)


def _run_d(xp2, xn2, c):
    return pl.pallas_call(
        _d_body,
        out_shape=jax.ShapeDtypeStruct((1, 1), jnp.float32),
    )(xp2, xn2, c)


def kernel(n_features, e_features, edge_index, W_apply_w, W_apply_b,
           W_edge_w, W_edge_b, disc_W):
    nf = jnp.concatenate(
        [n_features.reshape(_N, _DIN),
         jnp.zeros((_NP - _N, _DIN), jnp.float32)], axis=0)
    ef = e_features.reshape(_E, _EDIM)
    src = edge_index[0]
    dst = edge_index[1]
    perm = jnp.asarray(_PERM)

    spos, sneg, din, dout = _run_sc_scatter(ef, src, dst, perm)

    wnT = W_apply_w[:, :_DIN].T
    weT = W_apply_w[:, _DIN:].T
    brow = W_apply_b.reshape(1, _H)
    beb_row = W_edge_b.reshape(1, _EOUT)
    h_pos, h_neg, u, c = _run_b1(nf, spos, sneg, din, dout,
                                 wnT, weT, brow, W_edge_w, beb_row, disc_W)

    tab = _run_b3(h_pos, h_neg, u)

    xp, xn = _run_sc_edge(src, dst, tab.reshape(4 * _NP))

    loss = _run_d(xp.reshape(_E // _DIN, _DIN), xn.reshape(_E // _DIN, _DIN),
                  c)
    return loss[0, 0]


# speedup vs baseline: 19.7671x; 1.1194x over previous
import numpy as np
import jax
import jax.numpy as jnp
from jax import lax
from jax.experimental import pallas as pl
from jax.experimental.pallas import tpu as pltpu
from jax.experimental.pallas import tpu_sc as plsc

_N = 10000
_E = 320000
_DIN = 128
_EDIM = 16
_H = 128
_EOUT = 256

_NC = 2
_NS = 16
_NW = _NC * _NS
_NP = 10240
_ROWS_PT = _NP // _NS
_EPW = _E // _NW
_BLK = 400
_NBLK = _EPW // _BLK
_TN = 640


def _make_perm():
    try:
        try:
            dev = jax.local_devices(backend="cpu")[0]
            with jax.default_device(dev):
                p = np.asarray(jax.random.permutation(jax.random.key(1), _E))
        except Exception:
            p = np.asarray(jax.random.permutation(jax.random.key(1), _E))
    except Exception:
        p = np.arange(_E)
    return p.astype(np.int32)


_PERM = _make_perm()


def _sc_mesh():
    return plsc.VectorSubcoreMesh(core_axis_name="c", subcore_axis_name="s",
                                  num_cores=_NC, num_subcores=_NS)


def _sc_scatter_body(efeat, src, dst, perm, zeros_h, ones_h,
                     spos_o, sneg_o, din_o, dout_o,
                     feat0, feat1, featp_v, ones_v, zeros_v,
                     src0, src1, dst0, dst1, perm0, perm1,
                     acc_pos, acc_neg, acc_din, acc_dout,
                     sem0, sem1, sem_g):
    cid = lax.axis_index("c")
    sid = lax.axis_index("s")
    wid = cid * _NS + sid

    pltpu.sync_copy(zeros_h, zeros_v)
    pltpu.sync_copy(ones_h, ones_v)

    row0 = sid * _ROWS_PT
    pltpu.sync_copy(zeros_v, acc_pos.at[pl.ds(row0, _ROWS_PT)])
    pltpu.sync_copy(zeros_v, acc_neg.at[pl.ds(row0, _ROWS_PT)])
    pltpu.sync_copy(zeros_v, acc_din.at[pl.ds(row0, _ROWS_PT)])
    pltpu.sync_copy(zeros_v, acc_dout.at[pl.ds(row0, _ROWS_PT)])
    plsc.subcore_barrier()

    def loads(b, fv, sv, dv, pv, sem):
        base = wid * _EPW + b * _BLK
        pltpu.async_copy(efeat.at[pl.ds(base, _BLK)], fv, sem)
        pltpu.async_copy(src.at[pl.ds(base, _BLK)], sv, sem)
        pltpu.async_copy(dst.at[pl.ds(base, _BLK)], dv, sem)
        pltpu.async_copy(perm.at[pl.ds(base, _BLK)], pv, sem)

    def wait_loads(b, fv, sv, dv, pv, sem):
        base = wid * _EPW + b * _BLK
        pltpu.make_async_copy(efeat.at[pl.ds(base, _BLK)], fv, sem).wait()
        pltpu.make_async_copy(src.at[pl.ds(base, _BLK)], sv, sem).wait()
        pltpu.make_async_copy(dst.at[pl.ds(base, _BLK)], dv, sem).wait()
        pltpu.make_async_copy(perm.at[pl.ds(base, _BLK)], pv, sem).wait()

    def consume(fv, sv, dv, pv):
        g = pltpu.async_copy(efeat.at[pv], featp_v, sem_g)
        pltpu.sync_copy(fv, acc_pos.at[dv], add=True)
        pltpu.sync_copy(ones_v, acc_din.at[dv], add=True)
        pltpu.sync_copy(ones_v, acc_dout.at[sv], add=True)
        g.wait()
        pltpu.sync_copy(featp_v, acc_neg.at[dv], add=True)

    loads(0, feat0, src0, dst0, perm0, sem0)

    def g_body(g, carry):
        b0 = g * 2
        wait_loads(b0, feat0, src0, dst0, perm0, sem0)
        loads(b0 + 1, feat1, src1, dst1, perm1, sem1)
        consume(feat0, src0, dst0, perm0)
        wait_loads(b0 + 1, feat1, src1, dst1, perm1, sem1)
        loads(b0 + 2, feat0, src0, dst0, perm0, sem0)
        consume(feat1, src1, dst1, perm1)
        return carry

    lax.fori_loop(0, (_NBLK - 1) // 2, g_body, 0)
    wait_loads(_NBLK - 1, feat0, src0, dst0, perm0, sem0)
    consume(feat0, src0, dst0, perm0)
    plsc.subcore_barrier()

    rows = pl.ds(row0, _ROWS_PT)
    pltpu.sync_copy(acc_pos.at[rows], spos_o.at[cid, rows])
    pltpu.sync_copy(acc_neg.at[rows], sneg_o.at[cid, rows])
    pltpu.sync_copy(acc_din.at[rows], din_o.at[cid, rows])
    pltpu.sync_copy(acc_dout.at[rows], dout_o.at[cid, rows])


def _run_sc_scatter(ef, src, dst, perm):
    acc = jax.ShapeDtypeStruct((_NC, _NP, _EDIM), jnp.float32)
    f = pl.kernel(
        _sc_scatter_body,
        out_type=(acc, acc, acc, acc),
        mesh=_sc_mesh(),
        compiler_params=pltpu.CompilerParams(use_tc_tiling_on_sc=False),
        scratch_types=[
            pltpu.VMEM((_BLK, _EDIM), jnp.float32),
            pltpu.VMEM((_BLK, _EDIM), jnp.float32),
            pltpu.VMEM((_BLK, _EDIM), jnp.float32),
            pltpu.VMEM((_BLK, _EDIM), jnp.float32),
            pltpu.VMEM((_ROWS_PT, _EDIM), jnp.float32),
            pltpu.VMEM((_BLK,), jnp.int32),
            pltpu.VMEM((_BLK,), jnp.int32),
            pltpu.VMEM((_BLK,), jnp.int32),
            pltpu.VMEM((_BLK,), jnp.int32),
            pltpu.VMEM((_BLK,), jnp.int32),
            pltpu.VMEM((_BLK,), jnp.int32),
            pltpu.VMEM_SHARED((_NP, _EDIM), jnp.float32),
            pltpu.VMEM_SHARED((_NP, _EDIM), jnp.float32),
            pltpu.VMEM_SHARED((_NP, _EDIM), jnp.float32),
            pltpu.VMEM_SHARED((_NP, _EDIM), jnp.float32),
            pltpu.SemaphoreType.DMA,
            pltpu.SemaphoreType.DMA,
            pltpu.SemaphoreType.DMA,
        ],
    )
    zeros_h = jnp.zeros((_ROWS_PT, _EDIM), jnp.float32)
    ones_h = jnp.ones((_BLK, _EDIM), jnp.float32)
    return f(ef, src, dst, perm, zeros_h, ones_h)


def _b1_body(nf_ref, spos_ref, sneg_ref, din_ref, dout_ref,
             wnT_ref, weT_ref, b_ref, wew_ref, beb_ref, dw_ref,
             hp_ref, hn_ref, u_ref, c_ref, acc_ref):
    i = pl.program_id(0)
    din = din_ref[0, :, 0:1] + din_ref[1, :, 0:1]
    dout = dout_ref[0, :, 0:1] + dout_ref[1, :, 0:1]
    inv = 1.0 / jnp.maximum(din, 1.0)
    sp = (spos_ref[0] + spos_ref[1]) * inv
    sn = (sneg_ref[0] + sneg_ref[1]) * inv
    base = jnp.dot(nf_ref[...], wnT_ref[...],
                   preferred_element_type=jnp.float32) + b_ref[...]
    weT = weT_ref[...]
    hp = jnp.maximum(
        base + jnp.dot(sp, weT, preferred_element_type=jnp.float32), 0.0)
    hn = jnp.maximum(
        base + jnp.dot(sn, weT, preferred_element_type=jnp.float32), 0.0)
    hp_ref[...] = hp
    hn_ref[...] = hn

    mask = ((lax.broadcasted_iota(jnp.int32, (_TN, 1), 0) + i * _TN)
            < _N).astype(jnp.float32)
    msrc_t = jnp.sum(hp * (dout * mask), axis=0, keepdims=True)
    mdst_t = jnp.sum(hp * (din * mask), axis=0, keepdims=True)

    @pl.when(i == 0)
    def _init():
        acc_ref[...] = jnp.zeros((2, _H), jnp.float32)

    acc_ref[0:1, :] += msrc_t
    acc_ref[1:2, :] += mdst_t

    @pl.when(i == (_NP // _TN) - 1)
    def _fin():
        msrc = acc_ref[0:1, :]
        mdst = acc_ref[1:2, :]
        wew = wew_ref[...]
        w1 = wew[:, :_H]
        w2 = wew[:, _H:]
        dims = (((1,), (1,)), ((), ()))
        me = (lax.dot_general(msrc, w1, dims,
                              preferred_element_type=jnp.float32)
              + lax.dot_general(mdst, w2, dims,
                                preferred_element_type=jnp.float32)
              ) * (1.0 / _E) + beb_ref[...]
        summ = jax.nn.sigmoid(me)
        ws = lax.dot_general(summ, dw_ref[...], dims,
                             preferred_element_type=jnp.float32)
        dims2 = (((1,), (0,)), ((), ()))
        u1 = lax.dot_general(ws, w1, dims2, preferred_element_type=jnp.float32)
        u2 = lax.dot_general(ws, w2, dims2, preferred_element_type=jnp.float32)
        u_ref[...] = jnp.concatenate([u1, u2], axis=0)
        c_ref[...] = jnp.sum(beb_ref[...] * ws).reshape(1, 1)


def _run_b1(nf, spos, sneg, din, dout, wnT, weT, brow, wew, beb_row, dw):
    grid = (_NP // _TN,)
    seg = pl.BlockSpec((2, _TN, _EDIM), lambda i: (0, i, 0))
    hout = jax.ShapeDtypeStruct((_NP, _H), jnp.float32)
    return pl.pallas_call(
        _b1_body,
        grid=grid,
        in_specs=[
            pl.BlockSpec((_TN, _DIN), lambda i: (i, 0)),
            seg, seg, seg, seg,
            pl.BlockSpec((_DIN, _H), lambda i: (0, 0)),
            pl.BlockSpec((_EDIM, _H), lambda i: (0, 0)),
            pl.BlockSpec((1, _H), lambda i: (0, 0)),
            pl.BlockSpec((_EOUT, _EOUT), lambda i: (0, 0)),
            pl.BlockSpec((1, _EOUT), lambda i: (0, 0)),
            pl.BlockSpec((_EOUT, _EOUT), lambda i: (0, 0)),
        ],
        out_specs=[pl.BlockSpec((_TN, _H), lambda i: (i, 0)),
                   pl.BlockSpec((_TN, _H), lambda i: (i, 0)),
                   pl.BlockSpec((2, _H), lambda i: (0, 0)),
                   pl.BlockSpec((1, 1), lambda i: (0, 0))],
        out_shape=[hout, hout,
                   jax.ShapeDtypeStruct((2, _H), jnp.float32),
                   jax.ShapeDtypeStruct((1, 1), jnp.float32)],
        scratch_shapes=[pltpu.VMEM((2, _H), jnp.float32)],
    )(nf, spos, sneg, din, dout, wnT, weT, brow, wew, beb_row, dw)


def _b3_body(hp_ref, hn_ref, u_ref, o_ref):
    u = u_ref[...]
    dims = (((1,), (1,)), ((), ()))
    pq_p = lax.dot_general(u, hp_ref[...], dims,
                           preferred_element_type=jnp.float32)
    pq_n = lax.dot_general(u, hn_ref[...], dims,
                           preferred_element_type=jnp.float32)
    o_ref[...] = jnp.concatenate([pq_p, pq_n], axis=0)


def _run_b3(h_pos, h_neg, u):
    grid = (_NP // _TN,)
    return pl.pallas_call(
        _b3_body,
        grid=grid,
        in_specs=[
            pl.BlockSpec((_TN, _H), lambda i: (i, 0)),
            pl.BlockSpec((_TN, _H), lambda i: (i, 0)),
            pl.BlockSpec((2, _H), lambda i: (0, 0)),
        ],
        out_specs=pl.BlockSpec((4, _TN), lambda i: (0, i)),
        out_shape=jax.ShapeDtypeStruct((4, _NP), jnp.float32),
    )(h_pos, h_neg, u)


def _sc_edge_body(src, dst, tab, xp_o, xn_o,
                  tab_v, src_v, dst_v, xp_v, xn_v):
    cid = lax.axis_index("c")
    sid = lax.axis_index("s")
    wid = cid * _NS + sid

    pltpu.sync_copy(tab, tab_v)
    base = wid * _EPW
    pltpu.sync_copy(src.at[pl.ds(base, _EPW)], src_v)
    pltpu.sync_copy(dst.at[pl.ds(base, _EPW)], dst_v)

    def it(i, carry):
        s = pl.ds(i * 16, 16)
        sv = src_v[s]
        dv = dst_v[s]
        xp_v[s] = (plsc.load_gather(tab_v, [sv])
                   + plsc.load_gather(tab_v, [dv + _NP]))
        xn_v[s] = (plsc.load_gather(tab_v, [sv + 2 * _NP])
                   + plsc.load_gather(tab_v, [dv + 3 * _NP]))
        return carry

    lax.fori_loop(0, _EPW // 16, it, 0)
    pltpu.sync_copy(xp_v, xp_o.at[pl.ds(base, _EPW)])
    pltpu.sync_copy(xn_v, xn_o.at[pl.ds(base, _EPW)])


def _run_sc_edge(src, dst, tab):
    out = jax.ShapeDtypeStruct((_E,), jnp.float32)
    f = pl.kernel(
        _sc_edge_body,
        out_type=(out, out),
        mesh=_sc_mesh(),
        compiler_params=pltpu.CompilerParams(use_tc_tiling_on_sc=False,
                                             needs_layout_passes=False),
        scratch_types=[
            pltpu.VMEM((4 * _NP,), jnp.float32),
            pltpu.VMEM((_EPW,), jnp.int32),
            pltpu.VMEM((_EPW,), jnp.int32),
            pltpu.VMEM((_EPW,), jnp.float32),
            pltpu.VMEM((_EPW,), jnp.float32),
        ],
    )
    return f(src, dst, tab)


def _d_body(xp_ref, xn_ref, c_ref, o_ref):
    c = c_ref[0, 0]
    lp = jnp.mean(jax.nn.softplus(-(xp_ref[...] + c)))
    ln = jnp.mean(jax.nn.softplus(xn_ref[...] + c))
    o_ref[...] = (lp + ln).reshape(1, 1)


def _run_d(xp2, xn2, c):
    return pl.pallas_call(
        _d_body,
        out_shape=jax.ShapeDtypeStruct((1, 1), jnp.float32),
    )(xp2, xn2, c)


def kernel(n_features, e_features, edge_index, W_apply_w, W_apply_b,
           W_edge_w, W_edge_b, disc_W):
    nf = jnp.concatenate(
        [n_features.reshape(_N, _DIN),
         jnp.zeros((_NP - _N, _DIN), jnp.float32)], axis=0)
    ef = e_features.reshape(_E, _EDIM)
    src = edge_index[0]
    dst = edge_index[1]
    perm = jnp.asarray(_PERM)

    spos, sneg, din, dout = _run_sc_scatter(ef, src, dst, perm)

    wnT = W_apply_w[:, :_DIN].T
    weT = W_apply_w[:, _DIN:].T
    brow = W_apply_b.reshape(1, _H)
    beb_row = W_edge_b.reshape(1, _EOUT)
    h_pos, h_neg, u, c = _run_b1(nf, spos, sneg, din, dout,
                                 wnT, weT, brow, W_edge_w, beb_row, disc_W)

    tab = _run_b3(h_pos, h_neg, u)

    xp, xn = _run_sc_edge(src, dst, tab.reshape(4 * _NP))

    loss = _run_d(xp.reshape(_E // _DIN, _DIN), xn.reshape(_E // _DIN, _DIN),
                  c)
    return loss[0, 0]
